# initial kernel scaffold (unmeasured)
import functools

import jax
import jax.numpy as jnp
from jax import lax
from jax.experimental import pallas as pl
from jax.experimental.pallas import tpu as pltpu

N_DEV = 16
B = 2
M = 128
F = 512
H = 4
DH = 64
SKV = 128
S = N_DEV * M


def kernel(x, Wq, K_ext, V_ext, Wo):
    def body(x_ref, wq_ref, k_ref, v_ref, wo_ref, out_ref,
             xg_ref, yacc_ref, rsbuf_ref,
             ag_ssem, ag_rsem, rs_ssem, rs_rsem):
        my = lax.axis_index("i")
        left = jnp.mod(my - 1 + N_DEV, N_DEV)
        right = jnp.mod(my + 1, N_DEV)

        barrier_sem = pltpu.get_barrier_semaphore()
        for nbr in (left, right):
            pl.semaphore_signal(
                barrier_sem, inc=1,
                device_id=(nbr,), device_id_type=pl.DeviceIdType.MESH,
            )
        pl.semaphore_wait(barrier_sem, 2)

        xg_ref[my] = x_ref[...]

        for h in range(N_DEV - 1):
            idx = jnp.mod(my - h + N_DEV, N_DEV)
            rdma = pltpu.make_async_remote_copy(
                src_ref=xg_ref.at[idx],
                dst_ref=xg_ref.at[idx],
                send_sem=ag_ssem.at[h],
                recv_sem=ag_rsem.at[h],
                device_id=(right,),
                device_id_type=pl.DeviceIdType.MESH,
            )
            rdma.start()
            rdma.wait()

        k_my = lax.dynamic_slice_in_dim(k_ref[...], my * H, H, axis=2)
        v_my = lax.dynamic_slice_in_dim(v_ref[...], my * H, H, axis=2)
        wq = wq_ref[...]
        wo = wo_ref[...]

        qb = lax.broadcasted_iota(jnp.int32, (S, SKV), 0) // 64
        kb = lax.broadcasted_iota(jnp.int32, (S, SKV), 1) // 64
        mask = (qb == kb) | ((kb % 4) == (qb % 4))
        keep = jnp.any(mask, axis=1, keepdims=True)

        for b in range(B):
            xb = xg_ref[:, b].reshape(S, F)
            q = jnp.dot(xb, wq, preferred_element_type=jnp.float32)
            ctxs = []
            for h in range(H):
                qh = q[:, h * DH:(h + 1) * DH]
                kh = k_my[b, :, h, :]
                vh = v_my[b, :, h, :]
                s = jnp.dot(qh, kh.T, preferred_element_type=jnp.float32)
                s = s * 0.125
                s = jnp.where(mask, s, -1e9)
                m = jnp.max(s, axis=-1, keepdims=True)
                e = jnp.exp(s - m)
                den = jnp.sum(e, axis=-1, keepdims=True)
                w = jnp.where(keep, e / den, 0.0)
                ctxs.append(jnp.dot(w, vh, preferred_element_type=jnp.float32))
            ctx = jnp.concatenate(ctxs, axis=-1)
            yp = jnp.dot(ctx, wo, preferred_element_type=jnp.float32)
            yacc_ref[:, b] = yp.reshape(N_DEV, M, F)

        for t in range(N_DEV - 1):
            s_idx = jnp.mod(my - 1 - t + 2 * N_DEV, N_DEV)
            rdma = pltpu.make_async_remote_copy(
                src_ref=yacc_ref.at[s_idx],
                dst_ref=rsbuf_ref.at[t],
                send_sem=rs_ssem.at[t],
                recv_sem=rs_rsem.at[t],
                device_id=(right,),
                device_id_type=pl.DeviceIdType.MESH,
            )
            rdma.start()
            rdma.wait()
            r_idx = jnp.mod(my - 2 - t + 2 * N_DEV, N_DEV)
            yacc_ref[r_idx] = yacc_ref[r_idx] + rsbuf_ref[t]

        out_ref[...] = yacc_ref[my]

        @functools.partial(pl.run_scoped, exit_sem=pltpu.SemaphoreType.REGULAR)
        def _(exit_sem):
            for nbr in (left, right):
                pl.semaphore_signal(
                    exit_sem, inc=1,
                    device_id=(nbr,), device_id_type=pl.DeviceIdType.MESH,
                )
            pl.semaphore_wait(exit_sem, 2)

    return pl.pallas_call(
        body,
        out_shape=jax.ShapeDtypeStruct((B, M, F), jnp.float32),
        in_specs=[pl.BlockSpec(memory_space=pltpu.VMEM)] * 5,
        out_specs=pl.BlockSpec(memory_space=pltpu.VMEM),
        scratch_shapes=[
            pltpu.VMEM((N_DEV, B, M, F), jnp.float32),
            pltpu.VMEM((N_DEV, B, M, F), jnp.float32),
            pltpu.VMEM((N_DEV - 1, B, M, F), jnp.float32),
            pltpu.SemaphoreType.DMA((N_DEV - 1,)),
            pltpu.SemaphoreType.DMA((N_DEV - 1,)),
            pltpu.SemaphoreType.DMA((N_DEV - 1,)),
            pltpu.SemaphoreType.DMA((N_DEV - 1,)),
        ],
        compiler_params=pltpu.CompilerParams(collective_id=0),
    )(x, Wq, K_ext, V_ext, Wo)


# baseline (device time: 248665 ns/iter reference)
import functools

import jax
import jax.numpy as jnp
from jax import lax
from jax.experimental import pallas as pl
from jax.experimental.pallas import tpu as pltpu

N_DEV = 16
B = 2
M = 128
F = 512
H = 4
DH = 64
SKV = 128
S = N_DEV * M


def kernel(x, Wq, K_ext, V_ext, Wo):
    my_pos = lax.axis_index("i")
    K_my = lax.dynamic_slice_in_dim(K_ext, my_pos * H, H, axis=2)
    V_my = lax.dynamic_slice_in_dim(V_ext, my_pos * H, H, axis=2)

    def body(x_ref, wq_ref, k_ref, v_ref, wo_ref, out_ref,
             xg_ref, yacc_ref, rsbuf_ref,
             ag_ssem, ag_rsem, rs_ssem, rs_rsem):
        my = lax.axis_index("i")
        left = jnp.mod(my - 1 + N_DEV, N_DEV)
        right = jnp.mod(my + 1, N_DEV)

        barrier_sem = pltpu.get_barrier_semaphore()
        for nbr in (left, right):
            pl.semaphore_signal(
                barrier_sem, inc=1,
                device_id=(nbr,), device_id_type=pl.DeviceIdType.MESH,
            )
        pl.semaphore_wait(barrier_sem, 2)

        xg_ref[my] = x_ref[...]

        for h in range(N_DEV - 1):
            idx = jnp.mod(my - h + N_DEV, N_DEV)
            rdma = pltpu.make_async_remote_copy(
                src_ref=xg_ref.at[idx],
                dst_ref=xg_ref.at[idx],
                send_sem=ag_ssem.at[h],
                recv_sem=ag_rsem.at[h],
                device_id=(right,),
                device_id_type=pl.DeviceIdType.MESH,
            )
            rdma.start()
            rdma.wait()

        k_my = k_ref[...]
        v_my = v_ref[...]
        wq = wq_ref[...]
        wo = wo_ref[...]

        qb = lax.broadcasted_iota(jnp.int32, (S, SKV), 0) // 64
        kb = lax.broadcasted_iota(jnp.int32, (S, SKV), 1) // 64
        mask = (qb == kb) | ((kb % 4) == (qb % 4))
        keep = jnp.any(mask, axis=1, keepdims=True)

        for b in range(B):
            xb = xg_ref[:, b].reshape(S, F)
            q = jnp.dot(xb, wq, preferred_element_type=jnp.float32)
            ctxs = []
            for h in range(H):
                qh = q[:, h * DH:(h + 1) * DH]
                kh = k_my[b, :, h, :]
                vh = v_my[b, :, h, :]
                s = jnp.dot(qh, kh.T, preferred_element_type=jnp.float32)
                s = s * 0.125
                s = jnp.where(mask, s, -1e9)
                m = jnp.max(s, axis=-1, keepdims=True)
                e = jnp.exp(s - m)
                den = jnp.sum(e, axis=-1, keepdims=True)
                w = jnp.where(keep, e / den, 0.0)
                ctxs.append(jnp.dot(w, vh, preferred_element_type=jnp.float32))
            ctx = jnp.concatenate(ctxs, axis=-1)
            yp = jnp.dot(ctx, wo, preferred_element_type=jnp.float32)
            yacc_ref[:, b] = yp.reshape(N_DEV, M, F)

        for t in range(N_DEV - 1):
            s_idx = jnp.mod(my - 1 - t + 2 * N_DEV, N_DEV)
            rdma = pltpu.make_async_remote_copy(
                src_ref=yacc_ref.at[s_idx],
                dst_ref=rsbuf_ref.at[t],
                send_sem=rs_ssem.at[t],
                recv_sem=rs_rsem.at[t],
                device_id=(right,),
                device_id_type=pl.DeviceIdType.MESH,
            )
            rdma.start()
            rdma.wait()
            r_idx = jnp.mod(my - 2 - t + 2 * N_DEV, N_DEV)
            yacc_ref[r_idx] = yacc_ref[r_idx] + rsbuf_ref[t]

        out_ref[...] = yacc_ref[my]

        @functools.partial(pl.run_scoped, exit_sem=pltpu.SemaphoreType.REGULAR)
        def _(exit_sem):
            for nbr in (left, right):
                pl.semaphore_signal(
                    exit_sem, inc=1,
                    device_id=(nbr,), device_id_type=pl.DeviceIdType.MESH,
                )
            pl.semaphore_wait(exit_sem, 2)

    return pl.pallas_call(
        body,
        out_shape=jax.ShapeDtypeStruct((B, M, F), jnp.float32),
        in_specs=[pl.BlockSpec(memory_space=pltpu.VMEM)] * 5,
        out_specs=pl.BlockSpec(memory_space=pltpu.VMEM),
        scratch_shapes=[
            pltpu.VMEM((N_DEV, B, M, F), jnp.float32),
            pltpu.VMEM((N_DEV, B, M, F), jnp.float32),
            pltpu.VMEM((N_DEV - 1, B, M, F), jnp.float32),
            pltpu.SemaphoreType.DMA((N_DEV - 1,)),
            pltpu.SemaphoreType.DMA((N_DEV - 1,)),
            pltpu.SemaphoreType.DMA((N_DEV - 1,)),
            pltpu.SemaphoreType.DMA((N_DEV - 1,)),
        ],
        compiler_params=pltpu.CompilerParams(collective_id=0),
    )(x, Wq, K_my, V_my, Wo)


# device time: 242493 ns/iter; 1.0255x vs baseline; 1.0255x over previous
import functools

import jax
import jax.numpy as jnp
from jax import lax
from jax.experimental import pallas as pl
from jax.experimental.pallas import tpu as pltpu

N_DEV = 16
NL = N_DEV // 2
B = 2
M = 128
F = 512
H = 4
DH = 64
SKV = 128
SL = NL * M


def kernel(x, Wq, K_ext, V_ext, Wo):
    my_pos = lax.axis_index("i")
    K_my = lax.dynamic_slice_in_dim(K_ext, my_pos * H, H, axis=2)
    V_my = lax.dynamic_slice_in_dim(V_ext, my_pos * H, H, axis=2)

    def body(x_ref, wq_ref, k_ref, v_ref, wo_ref, out_ref,
             xg_ref, yacc_ref, rsbuf_ref,
             ag_ssem, ag_rsem, rs_ssem, rs_rsem):
        my = lax.axis_index("i")
        left = jnp.mod(my - 1 + N_DEV, N_DEV)
        right = jnp.mod(my + 1, N_DEV)
        i_am_live = jnp.mod(my, 2) == 0

        barrier_sem = pltpu.get_barrier_semaphore()
        for nbr in (left, right):
            pl.semaphore_signal(
                barrier_sem, inc=1,
                device_id=(nbr,), device_id_type=pl.DeviceIdType.MESH,
            )
        pl.semaphore_wait(barrier_sem, 2)

        @pl.when(i_am_live)
        def _():
            xg_ref[my // 2] = x_ref[...]

        for h in range(N_DEV - 1):
            send_idx = jnp.mod(my - h + N_DEV, N_DEV)
            recv_idx = jnp.mod(my - 1 - h + N_DEV, N_DEV)
            live_send = jnp.mod(send_idx, 2) == 0

            send_rdma = pltpu.make_async_remote_copy(
                src_ref=xg_ref.at[send_idx // 2],
                dst_ref=xg_ref.at[send_idx // 2],
                send_sem=ag_ssem.at[h],
                recv_sem=ag_rsem.at[h],
                device_id=(right,),
                device_id_type=pl.DeviceIdType.MESH,
            )
            recv_rdma = pltpu.make_async_remote_copy(
                src_ref=xg_ref.at[recv_idx // 2],
                dst_ref=xg_ref.at[recv_idx // 2],
                send_sem=ag_ssem.at[h],
                recv_sem=ag_rsem.at[h],
                device_id=(right,),
                device_id_type=pl.DeviceIdType.MESH,
            )

            @pl.when(live_send)
            def _():
                send_rdma.start()
                send_rdma.wait_send()

            @pl.when(jnp.logical_not(live_send))
            def _():
                recv_rdma.wait_recv()

        k_my = k_ref[...]
        v_my = v_ref[...]
        wq = wq_ref[...]
        wo = wo_ref[...]

        qg = lax.broadcasted_iota(jnp.int32, (SL, SKV), 0) // 64 % 2
        kbm = lax.broadcasted_iota(jnp.int32, (SL, SKV), 1) // 64
        mask = qg == kbm

        for b in range(B):
            xb = xg_ref[:, b].reshape(SL, F)
            q = jnp.dot(xb, wq, preferred_element_type=jnp.float32)
            ctxs = []
            for h in range(H):
                qh = q[:, h * DH:(h + 1) * DH]
                kh = k_my[b, :, h, :]
                vh = v_my[b, :, h, :]
                s = jnp.dot(qh, kh.T, preferred_element_type=jnp.float32)
                s = s * 0.125
                s = jnp.where(mask, s, -1e9)
                m = jnp.max(s, axis=-1, keepdims=True)
                e = jnp.exp(s - m)
                den = jnp.sum(e, axis=-1, keepdims=True)
                ctxs.append(jnp.dot(e / den, vh,
                                    preferred_element_type=jnp.float32))
            ctx = jnp.concatenate(ctxs, axis=-1)
            yp = jnp.dot(ctx, wo, preferred_element_type=jnp.float32)
            yacc_ref[:, b] = yp.reshape(NL, M, F)

        for t in range(N_DEV - 1):
            c_s = jnp.mod(my - 1 - t + 2 * N_DEV, N_DEV)
            c_r = jnp.mod(my - 2 - t + 2 * N_DEV, N_DEV)
            live_s = jnp.mod(c_s, 2) == 0

            send_rdma = pltpu.make_async_remote_copy(
                src_ref=yacc_ref.at[c_s // 2],
                dst_ref=rsbuf_ref.at[t // 2],
                send_sem=rs_ssem.at[t],
                recv_sem=rs_rsem.at[t],
                device_id=(right,),
                device_id_type=pl.DeviceIdType.MESH,
            )
            recv_rdma = pltpu.make_async_remote_copy(
                src_ref=rsbuf_ref.at[t // 2],
                dst_ref=rsbuf_ref.at[t // 2],
                send_sem=rs_ssem.at[t],
                recv_sem=rs_rsem.at[t],
                device_id=(right,),
                device_id_type=pl.DeviceIdType.MESH,
            )

            @pl.when(live_s)
            def _():
                send_rdma.start()
                send_rdma.wait_send()

            @pl.when(jnp.logical_not(live_s))
            def _():
                recv_rdma.wait_recv()
                yacc_ref[c_r // 2] = yacc_ref[c_r // 2] + rsbuf_ref[t // 2]

        @pl.when(i_am_live)
        def _():
            out_ref[...] = yacc_ref[my // 2]

        @pl.when(jnp.logical_not(i_am_live))
        def _():
            out_ref[...] = jnp.zeros((B, M, F), jnp.float32)

        @functools.partial(pl.run_scoped, exit_sem=pltpu.SemaphoreType.REGULAR)
        def _(exit_sem):
            for nbr in (left, right):
                pl.semaphore_signal(
                    exit_sem, inc=1,
                    device_id=(nbr,), device_id_type=pl.DeviceIdType.MESH,
                )
            pl.semaphore_wait(exit_sem, 2)

    return pl.pallas_call(
        body,
        out_shape=jax.ShapeDtypeStruct((B, M, F), jnp.float32),
        in_specs=[pl.BlockSpec(memory_space=pltpu.VMEM)] * 5,
        out_specs=pl.BlockSpec(memory_space=pltpu.VMEM),
        scratch_shapes=[
            pltpu.VMEM((NL, B, M, F), jnp.float32),
            pltpu.VMEM((NL, B, M, F), jnp.float32),
            pltpu.VMEM((NL, B, M, F), jnp.float32),
            pltpu.SemaphoreType.DMA((N_DEV - 1,)),
            pltpu.SemaphoreType.DMA((N_DEV - 1,)),
            pltpu.SemaphoreType.DMA((N_DEV - 1,)),
            pltpu.SemaphoreType.DMA((N_DEV - 1,)),
        ],
        compiler_params=pltpu.CompilerParams(collective_id=0),
    )(x, Wq, K_my, V_my, Wo)


# device time: 138807 ns/iter; 1.7914x vs baseline; 1.7470x over previous
import functools

import jax
import jax.numpy as jnp
from jax import lax
from jax.experimental import pallas as pl
from jax.experimental.pallas import tpu as pltpu

N_DEV = 16
NL = N_DEV // 2
B = 2
M = 128
F = 512
H = 4
DH = 64
SKV = 128


def kernel(x, Wq, K_ext, V_ext, Wo):
    my_pos = lax.axis_index("i")
    K_my = lax.dynamic_slice_in_dim(K_ext, my_pos * H, H, axis=2)
    V_my = lax.dynamic_slice_in_dim(V_ext, my_pos * H, H, axis=2)

    def body(x_ref, wq_ref, k_ref, v_ref, wo_ref, out_ref,
             xg_ref, yacc_ref, rsbuf_ref,
             blk_ssem, blk_rsem, chk_ssem, chk_rsem):
        my = lax.axis_index("i")
        left = jnp.mod(my - 1 + N_DEV, N_DEV)
        right = jnp.mod(my + 1, N_DEV)
        i_am_live = jnp.mod(my, 2) == 0

        barrier_sem = pltpu.get_barrier_semaphore()
        for nbr in (left, right):
            pl.semaphore_signal(
                barrier_sem, inc=1,
                device_id=(nbr,), device_id_type=pl.DeviceIdType.MESH,
            )
        pl.semaphore_wait(barrier_sem, 2)

        @pl.when(i_am_live)
        def _():
            xg_ref[my // 2] = x_ref[...]

        k_my = k_ref[...]
        v_my = v_ref[...]
        wq = wq_ref[...]
        wo = wo_ref[...]

        qg = lax.broadcasted_iota(jnp.int32, (M, SKV), 0) // 64
        kbm = lax.broadcasted_iota(jnp.int32, (M, SKV), 1) // 64
        mask = qg == kbm

        def compute_partial(slot):
            for b in range(B):
                xb = xg_ref[slot, b]
                q = jnp.dot(xb, wq, preferred_element_type=jnp.float32)
                ctxs = []
                for h in range(H):
                    qh = q[:, h * DH:(h + 1) * DH]
                    s = jnp.dot(qh, k_my[b, :, h, :].T,
                                preferred_element_type=jnp.float32)
                    s = jnp.where(mask, s * 0.125, -1e9)
                    m = jnp.max(s, axis=-1, keepdims=True)
                    e = jnp.exp(s - m)
                    den = jnp.sum(e, axis=-1, keepdims=True)
                    ctxs.append(jnp.dot(e / den, v_my[b, :, h, :],
                                        preferred_element_type=jnp.float32))
                ctx = jnp.concatenate(ctxs, axis=-1)
                yacc_ref[slot, b] = jnp.dot(
                    ctx, wo, preferred_element_type=jnp.float32)

        for g in range(N_DEV + 1):
            bs = jnp.mod(my - g + 2 * N_DEV, N_DEV)
            cs = jnp.mod(my + 1 - g + 2 * N_DEV, N_DEV)
            tick_a = jnp.mod(bs, 2) == 0
            br = jnp.mod(my - 1 - g + 2 * N_DEV, N_DEV)

            if g < N_DEV - 1:
                blk_send = pltpu.make_async_remote_copy(
                    src_ref=xg_ref.at[bs // 2], dst_ref=xg_ref.at[bs // 2],
                    send_sem=blk_ssem.at[g], recv_sem=blk_rsem.at[g],
                    device_id=(right,), device_id_type=pl.DeviceIdType.MESH,
                )
                blk_recv = pltpu.make_async_remote_copy(
                    src_ref=xg_ref.at[br // 2], dst_ref=xg_ref.at[br // 2],
                    send_sem=blk_ssem.at[g], recv_sem=blk_rsem.at[g],
                    device_id=(right,), device_id_type=pl.DeviceIdType.MESH,
                )
            if 2 <= g:
                chk_send = pltpu.make_async_remote_copy(
                    src_ref=yacc_ref.at[cs // 2],
                    dst_ref=rsbuf_ref.at[(g - 2) // 2],
                    send_sem=chk_ssem.at[g - 2], recv_sem=chk_rsem.at[g - 2],
                    device_id=(right,), device_id_type=pl.DeviceIdType.MESH,
                )
                chk_recv = pltpu.make_async_remote_copy(
                    src_ref=rsbuf_ref.at[(g - 2) // 2],
                    dst_ref=rsbuf_ref.at[(g - 2) // 2],
                    send_sem=chk_ssem.at[g - 2], recv_sem=chk_rsem.at[g - 2],
                    device_id=(right,), device_id_type=pl.DeviceIdType.MESH,
                )

            if g < N_DEV - 1:
                @pl.when(tick_a)
                def _():
                    blk_send.start()
            if 2 <= g:
                @pl.when(jnp.logical_not(tick_a))
                def _():
                    chk_send.start()

            if g < N_DEV:
                @pl.when(tick_a)
                def _():
                    compute_partial(bs // 2)

            if g < N_DEV - 1:
                @pl.when(jnp.logical_not(tick_a))
                def _():
                    blk_recv.wait_recv()
            if 2 <= g:
                @pl.when(tick_a)
                def _():
                    chk_recv.wait_recv()
                    yacc_ref[bs // 2] = (
                        yacc_ref[bs // 2] + rsbuf_ref[(g - 2) // 2])
            if g < N_DEV - 1:
                @pl.when(tick_a)
                def _():
                    blk_send.wait_send()
            if 2 <= g:
                @pl.when(jnp.logical_not(tick_a))
                def _():
                    chk_send.wait_send()

        @pl.when(i_am_live)
        def _():
            out_ref[...] = yacc_ref[my // 2]

        @pl.when(jnp.logical_not(i_am_live))
        def _():
            out_ref[...] = jnp.zeros((B, M, F), jnp.float32)

        @functools.partial(pl.run_scoped, exit_sem=pltpu.SemaphoreType.REGULAR)
        def _(exit_sem):
            for nbr in (left, right):
                pl.semaphore_signal(
                    exit_sem, inc=1,
                    device_id=(nbr,), device_id_type=pl.DeviceIdType.MESH,
                )
            pl.semaphore_wait(exit_sem, 2)

    return pl.pallas_call(
        body,
        out_shape=jax.ShapeDtypeStruct((B, M, F), jnp.float32),
        in_specs=[pl.BlockSpec(memory_space=pltpu.VMEM)] * 5,
        out_specs=pl.BlockSpec(memory_space=pltpu.VMEM),
        scratch_shapes=[
            pltpu.VMEM((NL, B, M, F), jnp.float32),
            pltpu.VMEM((NL, B, M, F), jnp.float32),
            pltpu.VMEM((NL, B, M, F), jnp.float32),
            pltpu.SemaphoreType.DMA((N_DEV - 1,)),
            pltpu.SemaphoreType.DMA((N_DEV - 1,)),
            pltpu.SemaphoreType.DMA((N_DEV - 1,)),
            pltpu.SemaphoreType.DMA((N_DEV - 1,)),
        ],
        compiler_params=pltpu.CompilerParams(collective_id=0),
    )(x, Wq, K_my, V_my, Wo)


# device time: 91205 ns/iter; 2.7264x vs baseline; 1.5219x over previous
import functools

import jax
import jax.numpy as jnp
from jax import lax
from jax.experimental import pallas as pl
from jax.experimental.pallas import tpu as pltpu

N_DEV = 16
NL = N_DEV // 2
B = 2
M = 128
F = 512
H = 4
DH = 64
SKV = 128


def kernel(x, Wq, K_ext, V_ext, Wo):
    my_pos = lax.axis_index("i")
    K_my = lax.dynamic_slice_in_dim(K_ext, my_pos * H, H, axis=2)
    V_my = lax.dynamic_slice_in_dim(V_ext, my_pos * H, H, axis=2)

    def body(x_ref, wq_ref, k_ref, v_ref, wo_ref, out_ref,
             xg_ref, yacc_ref, rsbuf_ref, ysnd_ref,
             blk_ssem, blk_rsem, chk_ssem, chk_rsem):
        my = lax.axis_index("i")
        left = jnp.mod(my - 1 + N_DEV, N_DEV)
        right = jnp.mod(my + 1, N_DEV)
        i_am_live = jnp.mod(my, 2) == 0

        barrier_sem = pltpu.get_barrier_semaphore()
        for nbr in (left, right):
            pl.semaphore_signal(
                barrier_sem, inc=1,
                device_id=(nbr,), device_id_type=pl.DeviceIdType.MESH,
            )
        pl.semaphore_wait(barrier_sem, 2)

        @pl.when(i_am_live)
        def _():
            xg_ref[my // 2] = x_ref[...].astype(jnp.bfloat16)

        k_my = k_ref[...]
        v_my = v_ref[...]
        wq = wq_ref[...]
        wo = wo_ref[...]

        qg = lax.broadcasted_iota(jnp.int32, (M, SKV), 0) // 64
        kbm = lax.broadcasted_iota(jnp.int32, (M, SKV), 1) // 64
        mask = qg == kbm

        def compute_partial(slot):
            for b in range(B):
                xb = xg_ref[slot, b].astype(jnp.float32)
                q = jnp.dot(xb, wq, preferred_element_type=jnp.float32)
                ctxs = []
                for h in range(H):
                    qh = q[:, h * DH:(h + 1) * DH]
                    s = jnp.dot(qh, k_my[b, :, h, :].T,
                                preferred_element_type=jnp.float32)
                    s = jnp.where(mask, s * 0.125, -1e9)
                    m = jnp.max(s, axis=-1, keepdims=True)
                    e = jnp.exp(s - m)
                    den = jnp.sum(e, axis=-1, keepdims=True)
                    ctxs.append(jnp.dot(e / den, v_my[b, :, h, :],
                                        preferred_element_type=jnp.float32))
                ctx = jnp.concatenate(ctxs, axis=-1)
                yacc_ref[slot, b] = jnp.dot(
                    ctx, wo, preferred_element_type=jnp.float32)

        for g in range(N_DEV + 1):
            bs = jnp.mod(my - g + 2 * N_DEV, N_DEV)
            cs = jnp.mod(my + 1 - g + 2 * N_DEV, N_DEV)
            tick_a = jnp.mod(bs, 2) == 0
            br = jnp.mod(my - 1 - g + 2 * N_DEV, N_DEV)

            if g < N_DEV - 1:
                blk_send = pltpu.make_async_remote_copy(
                    src_ref=xg_ref.at[bs // 2], dst_ref=xg_ref.at[bs // 2],
                    send_sem=blk_ssem.at[g], recv_sem=blk_rsem.at[g],
                    device_id=(right,), device_id_type=pl.DeviceIdType.MESH,
                )
                blk_recv = pltpu.make_async_remote_copy(
                    src_ref=xg_ref.at[br // 2], dst_ref=xg_ref.at[br // 2],
                    send_sem=blk_ssem.at[g], recv_sem=blk_rsem.at[g],
                    device_id=(right,), device_id_type=pl.DeviceIdType.MESH,
                )
            if 2 <= g:
                chk_send = pltpu.make_async_remote_copy(
                    src_ref=ysnd_ref,
                    dst_ref=rsbuf_ref.at[(g - 2) // 2],
                    send_sem=chk_ssem.at[g - 2], recv_sem=chk_rsem.at[g - 2],
                    device_id=(right,), device_id_type=pl.DeviceIdType.MESH,
                )
                chk_recv = pltpu.make_async_remote_copy(
                    src_ref=rsbuf_ref.at[(g - 2) // 2],
                    dst_ref=rsbuf_ref.at[(g - 2) // 2],
                    send_sem=chk_ssem.at[g - 2], recv_sem=chk_rsem.at[g - 2],
                    device_id=(right,), device_id_type=pl.DeviceIdType.MESH,
                )

            if g < N_DEV - 1:
                @pl.when(tick_a)
                def _():
                    blk_send.start()
            if 2 <= g:
                @pl.when(jnp.logical_not(tick_a))
                def _():
                    ysnd_ref[...] = yacc_ref[cs // 2].astype(jnp.bfloat16)
                    chk_send.start()

            if g < N_DEV:
                @pl.when(tick_a)
                def _():
                    compute_partial(bs // 2)

            if g < N_DEV - 1:
                @pl.when(jnp.logical_not(tick_a))
                def _():
                    blk_recv.wait_recv()
            if 2 <= g:
                @pl.when(tick_a)
                def _():
                    chk_recv.wait_recv()
                    yacc_ref[bs // 2] = (
                        yacc_ref[bs // 2]
                        + rsbuf_ref[(g - 2) // 2].astype(jnp.float32))
            if g < N_DEV - 1:
                @pl.when(tick_a)
                def _():
                    blk_send.wait_send()
            if 2 <= g:
                @pl.when(jnp.logical_not(tick_a))
                def _():
                    chk_send.wait_send()

        @pl.when(i_am_live)
        def _():
            out_ref[...] = yacc_ref[my // 2]

        @pl.when(jnp.logical_not(i_am_live))
        def _():
            out_ref[...] = jnp.zeros((B, M, F), jnp.float32)

        @functools.partial(pl.run_scoped, exit_sem=pltpu.SemaphoreType.REGULAR)
        def _(exit_sem):
            for nbr in (left, right):
                pl.semaphore_signal(
                    exit_sem, inc=1,
                    device_id=(nbr,), device_id_type=pl.DeviceIdType.MESH,
                )
            pl.semaphore_wait(exit_sem, 2)

    return pl.pallas_call(
        body,
        out_shape=jax.ShapeDtypeStruct((B, M, F), jnp.float32),
        in_specs=[pl.BlockSpec(memory_space=pltpu.VMEM)] * 5,
        out_specs=pl.BlockSpec(memory_space=pltpu.VMEM),
        scratch_shapes=[
            pltpu.VMEM((NL, B, M, F), jnp.bfloat16),
            pltpu.VMEM((NL, B, M, F), jnp.float32),
            pltpu.VMEM((NL, B, M, F), jnp.bfloat16),
            pltpu.VMEM((B, M, F), jnp.bfloat16),
            pltpu.SemaphoreType.DMA((N_DEV - 1,)),
            pltpu.SemaphoreType.DMA((N_DEV - 1,)),
            pltpu.SemaphoreType.DMA((N_DEV - 1,)),
            pltpu.SemaphoreType.DMA((N_DEV - 1,)),
        ],
        compiler_params=pltpu.CompilerParams(collective_id=0),
    )(x, Wq, K_my, V_my, Wo)


# device time: 90615 ns/iter; 2.7442x vs baseline; 1.0065x over previous
import functools

import jax
import jax.numpy as jnp
from jax import lax
from jax.experimental import pallas as pl
from jax.experimental.pallas import tpu as pltpu

N_DEV = 16
NL = N_DEV // 2
B = 2
M = 128
MH = 64
F = 512
H = 4
DH = 64
NT = N_DEV + 1


def kernel(x, Wq, K_ext, V_ext, Wo):
    my_pos = lax.axis_index("i")
    K_my = lax.dynamic_slice_in_dim(K_ext, my_pos * H, H, axis=2)
    V_my = lax.dynamic_slice_in_dim(V_ext, my_pos * H, H, axis=2)

    def body(x_ref, wq_ref, k_ref, v_ref, wo_ref, out_ref,
             xg_r, xg_l, yacc_r, yacc_l, rsbuf_r, rsbuf_l, ysnd_r, ysnd_l,
             blk_ssem_r, blk_rsem_r, chk_ssem_r, chk_rsem_r,
             blk_ssem_l, blk_rsem_l, chk_ssem_l, chk_rsem_l):
        my = lax.axis_index("i")
        left = jnp.mod(my - 1 + N_DEV, N_DEV)
        right = jnp.mod(my + 1, N_DEV)
        i_am_live = jnp.mod(my, 2) == 0

        barrier_sem = pltpu.get_barrier_semaphore()
        for nbr in (left, right):
            pl.semaphore_signal(
                barrier_sem, inc=1,
                device_id=(nbr,), device_id_type=pl.DeviceIdType.MESH,
            )
        pl.semaphore_wait(barrier_sem, 2)

        @pl.when(i_am_live)
        def _():
            xg_r[my // 2] = x_ref[:, 0:MH, :].astype(jnp.bfloat16)
            xg_l[my // 2] = x_ref[:, MH:M, :].astype(jnp.bfloat16)

        kb = k_ref[...].astype(jnp.bfloat16)
        vb = v_ref[...].astype(jnp.bfloat16)
        wqb = wq_ref[...].astype(jnp.bfloat16)
        wob = wo_ref[...].astype(jnp.bfloat16)

        def compute_partials(slot_r, slot_l):
            for b in range(B):
                xcat = jnp.concatenate([xg_r[slot_r, b], xg_l[slot_l, b]],
                                       axis=0)
                q = jnp.dot(xcat, wqb, preferred_element_type=jnp.float32)
                qbf = q.astype(jnp.bfloat16)
                ctxs = []
                for h in range(H):
                    qh_r = qbf[0:MH, h * DH:(h + 1) * DH]
                    qh_l = qbf[MH:M, h * DH:(h + 1) * DH]
                    for qh, j0 in ((qh_r, 0), (qh_l, MH)):
                        s = jnp.dot(qh, kb[b, j0:j0 + MH, h, :].T,
                                    preferred_element_type=jnp.float32)
                        s = s * 0.125
                        m = jnp.max(s, axis=-1, keepdims=True)
                        e = jnp.exp(s - m)
                        den = jnp.sum(e, axis=-1, keepdims=True)
                        w = (e / den).astype(jnp.bfloat16)
                        ctxs.append((h, j0, jnp.dot(
                            w, vb[b, j0:j0 + MH, h, :],
                            preferred_element_type=jnp.float32)))
                ctx_r = jnp.concatenate(
                    [c for h, j0, c in ctxs if j0 == 0], axis=-1)
                ctx_l = jnp.concatenate(
                    [c for h, j0, c in ctxs if j0 == MH], axis=-1)
                ycat = jnp.dot(
                    jnp.concatenate([ctx_r, ctx_l], axis=0).astype(
                        jnp.bfloat16),
                    wob, preferred_element_type=jnp.float32)
                yacc_r[slot_r, b] = ycat[0:MH, :]
                yacc_l[slot_l, b] = ycat[MH:M, :]

        for g in range(NT):
            bs_r = jnp.mod(my - g + 2 * N_DEV, N_DEV)
            cs_r = jnp.mod(my + 1 - g + 2 * N_DEV, N_DEV)
            br_r = jnp.mod(my - 1 - g + 2 * N_DEV, N_DEV)
            bs_l = jnp.mod(my + g, N_DEV)
            cs_l = jnp.mod(my - 1 + g, N_DEV)
            br_l = jnp.mod(my + 1 + g, N_DEV)
            tick_a = jnp.mod(bs_r, 2) == 0

            dirs = []
            if g < N_DEV - 1:
                dirs = [
                    (pltpu.make_async_remote_copy(
                        src_ref=xg_r.at[bs_r // 2], dst_ref=xg_r.at[bs_r // 2],
                        send_sem=blk_ssem_r.at[g], recv_sem=blk_rsem_r.at[g],
                        device_id=(right,),
                        device_id_type=pl.DeviceIdType.MESH),
                     pltpu.make_async_remote_copy(
                        src_ref=xg_r.at[br_r // 2], dst_ref=xg_r.at[br_r // 2],
                        send_sem=blk_ssem_r.at[g], recv_sem=blk_rsem_r.at[g],
                        device_id=(right,),
                        device_id_type=pl.DeviceIdType.MESH)),
                    (pltpu.make_async_remote_copy(
                        src_ref=xg_l.at[bs_l // 2], dst_ref=xg_l.at[bs_l // 2],
                        send_sem=blk_ssem_l.at[g], recv_sem=blk_rsem_l.at[g],
                        device_id=(left,),
                        device_id_type=pl.DeviceIdType.MESH),
                     pltpu.make_async_remote_copy(
                        src_ref=xg_l.at[br_l // 2], dst_ref=xg_l.at[br_l // 2],
                        send_sem=blk_ssem_l.at[g], recv_sem=blk_rsem_l.at[g],
                        device_id=(left,),
                        device_id_type=pl.DeviceIdType.MESH)),
                ]
            chks = []
            if 2 <= g:
                slot = (g - 2) // 2
                chks = [
                    (cs_r, yacc_r, ysnd_r, pltpu.make_async_remote_copy(
                        src_ref=ysnd_r, dst_ref=rsbuf_r.at[slot],
                        send_sem=chk_ssem_r.at[g - 2],
                        recv_sem=chk_rsem_r.at[g - 2],
                        device_id=(right,),
                        device_id_type=pl.DeviceIdType.MESH),
                     bs_r, rsbuf_r, pltpu.make_async_remote_copy(
                        src_ref=rsbuf_r.at[slot], dst_ref=rsbuf_r.at[slot],
                        send_sem=chk_ssem_r.at[g - 2],
                        recv_sem=chk_rsem_r.at[g - 2],
                        device_id=(right,),
                        device_id_type=pl.DeviceIdType.MESH)),
                    (cs_l, yacc_l, ysnd_l, pltpu.make_async_remote_copy(
                        src_ref=ysnd_l, dst_ref=rsbuf_l.at[slot],
                        send_sem=chk_ssem_l.at[g - 2],
                        recv_sem=chk_rsem_l.at[g - 2],
                        device_id=(left,),
                        device_id_type=pl.DeviceIdType.MESH),
                     bs_l, rsbuf_l, pltpu.make_async_remote_copy(
                        src_ref=rsbuf_l.at[slot], dst_ref=rsbuf_l.at[slot],
                        send_sem=chk_ssem_l.at[g - 2],
                        recv_sem=chk_rsem_l.at[g - 2],
                        device_id=(left,),
                        device_id_type=pl.DeviceIdType.MESH)),
                ]

            if dirs:
                @pl.when(tick_a)
                def _():
                    for blk_send, _blk_recv in dirs:
                        blk_send.start()
            if chks:
                @pl.when(jnp.logical_not(tick_a))
                def _():
                    for cs, yacc, ysnd, chk_send, _, _, _ in chks:
                        ysnd[...] = yacc[cs // 2].astype(jnp.bfloat16)
                        chk_send.start()

            if g < N_DEV:
                @pl.when(tick_a)
                def _():
                    compute_partials(bs_r // 2, bs_l // 2)

            if dirs:
                @pl.when(jnp.logical_not(tick_a))
                def _():
                    for _blk_send, blk_recv in dirs:
                        blk_recv.wait_recv()
            if chks:
                @pl.when(tick_a)
                def _():
                    slot = (g - 2) // 2
                    for _, yacc, _, _, cr, rsbuf, chk_recv in chks:
                        chk_recv.wait_recv()
                        yacc[cr // 2] = (
                            yacc[cr // 2] + rsbuf[slot].astype(jnp.float32))
            if dirs:
                @pl.when(tick_a)
                def _():
                    for blk_send, _blk_recv in dirs:
                        blk_send.wait_send()
            if chks:
                @pl.when(jnp.logical_not(tick_a))
                def _():
                    for _, _, _, chk_send, _, _, _ in chks:
                        chk_send.wait_send()

        @pl.when(i_am_live)
        def _():
            out_ref[:, 0:MH, :] = yacc_r[my // 2]
            out_ref[:, MH:M, :] = yacc_l[my // 2]

        @pl.when(jnp.logical_not(i_am_live))
        def _():
            out_ref[...] = jnp.zeros((B, M, F), jnp.float32)

        @functools.partial(pl.run_scoped, exit_sem=pltpu.SemaphoreType.REGULAR)
        def _(exit_sem):
            for nbr in (left, right):
                pl.semaphore_signal(
                    exit_sem, inc=1,
                    device_id=(nbr,), device_id_type=pl.DeviceIdType.MESH,
                )
            pl.semaphore_wait(exit_sem, 2)

    sem15 = pltpu.SemaphoreType.DMA((N_DEV - 1,))
    return pl.pallas_call(
        body,
        out_shape=jax.ShapeDtypeStruct((B, M, F), jnp.float32),
        in_specs=[pl.BlockSpec(memory_space=pltpu.VMEM)] * 5,
        out_specs=pl.BlockSpec(memory_space=pltpu.VMEM),
        scratch_shapes=[
            pltpu.VMEM((NL, B, MH, F), jnp.bfloat16),
            pltpu.VMEM((NL, B, MH, F), jnp.bfloat16),
            pltpu.VMEM((NL, B, MH, F), jnp.float32),
            pltpu.VMEM((NL, B, MH, F), jnp.float32),
            pltpu.VMEM((NL, B, MH, F), jnp.bfloat16),
            pltpu.VMEM((NL, B, MH, F), jnp.bfloat16),
            pltpu.VMEM((B, MH, F), jnp.bfloat16),
            pltpu.VMEM((B, MH, F), jnp.bfloat16),
            sem15, sem15, sem15, sem15,
            sem15, sem15, sem15, sem15,
        ],
        compiler_params=pltpu.CompilerParams(collective_id=0),
    )(x, Wq, K_my, V_my, Wo)


# device time: 70647 ns/iter; 3.5198x vs baseline; 1.2826x over previous
import functools

import jax
import jax.numpy as jnp
from jax import lax
from jax.experimental import pallas as pl
from jax.experimental.pallas import tpu as pltpu

N_DEV = 16
NL = N_DEV // 2
B = 2
M = 128
F = 512
H = 4
DH = 64
SKV = 128
NT = 9


def kernel(x, Wq, K_ext, V_ext, Wo):
    my_pos = lax.axis_index("i")
    K_my = lax.dynamic_slice_in_dim(K_ext, my_pos * H, H, axis=2)
    V_my = lax.dynamic_slice_in_dim(V_ext, my_pos * H, H, axis=2)

    def body(x_ref, wq_ref, k_ref, v_ref, wo_ref, out_ref,
             xg_ref, yacc_ref, rsbuf_r, rsbuf_l, ysnd_r, ysnd_l,
             yfin_r, yfin_l,
             blk_ssem_r, blk_rsem_r, chk_ssem_r, chk_rsem_r,
             blk_ssem_l, blk_rsem_l, chk_ssem_l, chk_rsem_l,
             fin_ssem_r, fin_rsem_r, fin_ssem_l, fin_rsem_l):
        my = lax.axis_index("i")
        left = jnp.mod(my - 1 + N_DEV, N_DEV)
        right = jnp.mod(my + 1, N_DEV)
        i_am_live = jnp.mod(my, 2) == 0

        barrier_sem = pltpu.get_barrier_semaphore()
        for nbr in (left, right):
            pl.semaphore_signal(
                barrier_sem, inc=1,
                device_id=(nbr,), device_id_type=pl.DeviceIdType.MESH,
            )
        pl.semaphore_wait(barrier_sem, 2)

        @pl.when(i_am_live)
        def _():
            xg_ref[my // 2] = x_ref[...].astype(jnp.bfloat16)

        kb = k_ref[...].astype(jnp.bfloat16)
        vb = v_ref[...].astype(jnp.bfloat16)
        wqb = wq_ref[...].astype(jnp.bfloat16)
        wob = wo_ref[...].astype(jnp.bfloat16)

        qg = lax.broadcasted_iota(jnp.int32, (M, SKV), 0) // 64
        kbm = lax.broadcasted_iota(jnp.int32, (M, SKV), 1) // 64
        mask = qg == kbm

        def compute_partial(slot):
            for b in range(B):
                xb = xg_ref[slot, b]
                q = jnp.dot(xb, wqb, preferred_element_type=jnp.float32)
                qbf = q.astype(jnp.bfloat16)
                ctxs = []
                for h in range(H):
                    qh = qbf[:, h * DH:(h + 1) * DH]
                    s = jnp.dot(qh, kb[b, :, h, :].T,
                                preferred_element_type=jnp.float32)
                    s = jnp.where(mask, s * 0.125, -1e9)
                    m = jnp.max(s, axis=-1, keepdims=True)
                    e = jnp.exp(s - m)
                    den = jnp.sum(e, axis=-1, keepdims=True)
                    w = (e / den).astype(jnp.bfloat16)
                    ctxs.append(jnp.dot(w, vb[b, :, h, :],
                                        preferred_element_type=jnp.float32))
                ctx = jnp.concatenate(ctxs, axis=-1).astype(jnp.bfloat16)
                yacc_ref[slot, b] = jnp.dot(
                    ctx, wob, preferred_element_type=jnp.float32)

        for g in range(NT):
            bs_r = jnp.mod(my - g + 2 * N_DEV, N_DEV)
            br_r = jnp.mod(my - 1 - g + 2 * N_DEV, N_DEV)
            cs_r = jnp.mod(my + 1 - g + 2 * N_DEV, N_DEV)
            bs_l = jnp.mod(my + g, N_DEV)
            br_l = jnp.mod(my + 1 + g, N_DEV)
            cs_l = jnp.mod(my - 1 + g + N_DEV, N_DEV)
            tick_a = jnp.mod(bs_r, 2) == 0

            blks = []
            if g < 8:
                blks.append((pltpu.make_async_remote_copy(
                    src_ref=xg_ref.at[bs_r // 2], dst_ref=xg_ref.at[bs_r // 2],
                    send_sem=blk_ssem_r.at[g], recv_sem=blk_rsem_r.at[g],
                    device_id=(right,), device_id_type=pl.DeviceIdType.MESH),
                    pltpu.make_async_remote_copy(
                    src_ref=xg_ref.at[br_r // 2], dst_ref=xg_ref.at[br_r // 2],
                    send_sem=blk_ssem_r.at[g], recv_sem=blk_rsem_r.at[g],
                    device_id=(right,), device_id_type=pl.DeviceIdType.MESH)))
            if g < 7:
                blks.append((pltpu.make_async_remote_copy(
                    src_ref=xg_ref.at[bs_l // 2], dst_ref=xg_ref.at[bs_l // 2],
                    send_sem=blk_ssem_l.at[g], recv_sem=blk_rsem_l.at[g],
                    device_id=(left,), device_id_type=pl.DeviceIdType.MESH),
                    pltpu.make_async_remote_copy(
                    src_ref=xg_ref.at[br_l // 2], dst_ref=xg_ref.at[br_l // 2],
                    send_sem=blk_ssem_l.at[g], recv_sem=blk_rsem_l.at[g],
                    device_id=(left,), device_id_type=pl.DeviceIdType.MESH)))

            chks = []
            if 2 <= g:
                slot_r = (g - 2) // 2
                chks.append((cs_r, ysnd_r, pltpu.make_async_remote_copy(
                    src_ref=ysnd_r, dst_ref=rsbuf_r.at[slot_r],
                    send_sem=chk_ssem_r.at[g - 2],
                    recv_sem=chk_rsem_r.at[g - 2],
                    device_id=(right,), device_id_type=pl.DeviceIdType.MESH),
                    bs_r, rsbuf_r, slot_r, pltpu.make_async_remote_copy(
                    src_ref=rsbuf_r.at[slot_r], dst_ref=rsbuf_r.at[slot_r],
                    send_sem=chk_ssem_r.at[g - 2],
                    recv_sem=chk_rsem_r.at[g - 2],
                    device_id=(right,), device_id_type=pl.DeviceIdType.MESH)))
            if 2 <= g < 8:
                slot_l = (g - 2) // 2
                chks.append((cs_l, ysnd_l, pltpu.make_async_remote_copy(
                    src_ref=ysnd_l, dst_ref=rsbuf_l.at[slot_l],
                    send_sem=chk_ssem_l.at[g - 2],
                    recv_sem=chk_rsem_l.at[g - 2],
                    device_id=(left,), device_id_type=pl.DeviceIdType.MESH),
                    bs_l, rsbuf_l, slot_l, pltpu.make_async_remote_copy(
                    src_ref=rsbuf_l.at[slot_l], dst_ref=rsbuf_l.at[slot_l],
                    send_sem=chk_ssem_l.at[g - 2],
                    recv_sem=chk_rsem_l.at[g - 2],
                    device_id=(left,), device_id_type=pl.DeviceIdType.MESH)))

            if blks:
                @pl.when(tick_a)
                def _():
                    for blk_send, _ in blks:
                        blk_send.start()
            if chks:
                @pl.when(jnp.logical_not(tick_a))
                def _():
                    for cs, ysnd, chk_send, _, _, _, _ in chks:
                        ysnd[...] = yacc_ref[cs // 2].astype(jnp.bfloat16)
                        chk_send.start()

            @pl.when(tick_a)
            def _():
                compute_partial(bs_r // 2)
                if 1 <= g < 8:
                    compute_partial(bs_l // 2)

            if blks:
                @pl.when(jnp.logical_not(tick_a))
                def _():
                    for _, blk_recv in blks:
                        blk_recv.wait_recv()
            if chks:
                @pl.when(tick_a)
                def _():
                    for _, _, _, cr, rsbuf, slot, chk_recv in chks:
                        chk_recv.wait_recv()
                        yacc_ref[cr // 2] = (
                            yacc_ref[cr // 2]
                            + rsbuf[slot].astype(jnp.float32))
            if blks:
                @pl.when(tick_a)
                def _():
                    for blk_send, _ in blks:
                        blk_send.wait_send()
            if chks:
                @pl.when(jnp.logical_not(tick_a))
                def _():
                    for _, _, chk_send, _, _, _, _ in chks:
                        chk_send.wait_send()

        fl_owner = jnp.mod(my + 7, N_DEV)
        fr_owner = jnp.mod(my + 8, N_DEV)
        fin_l_send = pltpu.make_async_remote_copy(
            src_ref=ysnd_l, dst_ref=yfin_l,
            send_sem=fin_ssem_l, recv_sem=fin_rsem_l,
            device_id=(fl_owner,), device_id_type=pl.DeviceIdType.MESH)
        fin_l_recv = pltpu.make_async_remote_copy(
            src_ref=ysnd_l, dst_ref=yfin_l,
            send_sem=fin_ssem_l, recv_sem=fin_rsem_l,
            device_id=(fl_owner,), device_id_type=pl.DeviceIdType.MESH)
        fin_r_send = pltpu.make_async_remote_copy(
            src_ref=ysnd_r, dst_ref=yfin_r,
            send_sem=fin_ssem_r, recv_sem=fin_rsem_r,
            device_id=(fr_owner,), device_id_type=pl.DeviceIdType.MESH)
        fin_r_recv = pltpu.make_async_remote_copy(
            src_ref=ysnd_r, dst_ref=yfin_r,
            send_sem=fin_ssem_r, recv_sem=fin_rsem_r,
            device_id=(fr_owner,), device_id_type=pl.DeviceIdType.MESH)

        @pl.when(jnp.logical_not(i_am_live))
        def _():
            ysnd_l[...] = yacc_ref[jnp.mod(my + 7, N_DEV) // 2].astype(jnp.bfloat16)
            fin_l_send.start()
            fin_l_send.wait_send()
            out_ref[...] = jnp.zeros((B, M, F), jnp.float32)

        @pl.when(i_am_live)
        def _():
            ysnd_r[...] = yacc_ref[jnp.mod(my - 8 + N_DEV, N_DEV) // 2].astype(jnp.bfloat16)
            fin_r_send.start()
            fin_l_recv.wait_recv()
            fin_r_recv.wait_recv()
            out_ref[...] = (yacc_ref[my // 2]
                            + yfin_l[...].astype(jnp.float32)
                            + yfin_r[...].astype(jnp.float32))
            fin_r_send.wait_send()

        @functools.partial(pl.run_scoped, exit_sem=pltpu.SemaphoreType.REGULAR)
        def _(exit_sem):
            for nbr in (left, right):
                pl.semaphore_signal(
                    exit_sem, inc=1,
                    device_id=(nbr,), device_id_type=pl.DeviceIdType.MESH,
                )
            pl.semaphore_wait(exit_sem, 2)

    return pl.pallas_call(
        body,
        out_shape=jax.ShapeDtypeStruct((B, M, F), jnp.float32),
        in_specs=[pl.BlockSpec(memory_space=pltpu.VMEM)] * 5,
        out_specs=pl.BlockSpec(memory_space=pltpu.VMEM),
        scratch_shapes=[
            pltpu.VMEM((NL, B, M, F), jnp.bfloat16),
            pltpu.VMEM((NL, B, M, F), jnp.float32),
            pltpu.VMEM((4, B, M, F), jnp.bfloat16),
            pltpu.VMEM((3, B, M, F), jnp.bfloat16),
            pltpu.VMEM((B, M, F), jnp.bfloat16),
            pltpu.VMEM((B, M, F), jnp.bfloat16),
            pltpu.VMEM((B, M, F), jnp.bfloat16),
            pltpu.VMEM((B, M, F), jnp.bfloat16),
            pltpu.SemaphoreType.DMA((8,)),
            pltpu.SemaphoreType.DMA((8,)),
            pltpu.SemaphoreType.DMA((7,)),
            pltpu.SemaphoreType.DMA((7,)),
            pltpu.SemaphoreType.DMA((7,)),
            pltpu.SemaphoreType.DMA((7,)),
            pltpu.SemaphoreType.DMA((6,)),
            pltpu.SemaphoreType.DMA((6,)),
            pltpu.SemaphoreType.DMA,
            pltpu.SemaphoreType.DMA,
            pltpu.SemaphoreType.DMA,
            pltpu.SemaphoreType.DMA,
        ],
        compiler_params=pltpu.CompilerParams(collective_id=0),
    )(x, Wq, K_my, V_my, Wo)


# device time: 66620 ns/iter; 3.7326x vs baseline; 1.0604x over previous
import functools

import jax
import jax.numpy as jnp
from jax import lax
from jax.experimental import pallas as pl
from jax.experimental.pallas import tpu as pltpu

N_DEV = 16
NL = N_DEV // 2
B = 2
M = 128
F = 512
H = 4
DH = 64
SKV = 128
NT = 9


def kernel(x, Wq, K_ext, V_ext, Wo):
    my_pos = lax.axis_index("i")
    K_my = lax.dynamic_slice_in_dim(K_ext, my_pos * H, H, axis=2)
    V_my = lax.dynamic_slice_in_dim(V_ext, my_pos * H, H, axis=2)

    def body(x_ref, wq_ref, k_ref, v_ref, wo_ref, out_ref,
             xg_ref, yacc_ref, rsbuf_r, rsbuf_l, ysnd_r, ysnd_l,
             yfin_r, yfin_l,
             blk_ssem_r, blk_rsem_r, chk_ssem_r, chk_rsem_r,
             blk_ssem_l, blk_rsem_l, chk_ssem_l, chk_rsem_l,
             fin_ssem_r, fin_rsem_r, fin_ssem_l, fin_rsem_l):
        my = lax.axis_index("i")
        left = jnp.mod(my - 1 + N_DEV, N_DEV)
        right = jnp.mod(my + 1, N_DEV)
        i_am_live = jnp.mod(my, 2) == 0

        barrier_sem = pltpu.get_barrier_semaphore()
        for nbr in (left, right):
            pl.semaphore_signal(
                barrier_sem, inc=1,
                device_id=(nbr,), device_id_type=pl.DeviceIdType.MESH,
            )
        pl.semaphore_wait(barrier_sem, 2)

        @pl.when(i_am_live)
        def _():
            xg_ref[my // 2] = x_ref[...].astype(jnp.bfloat16)

        kb = k_ref[...].astype(jnp.bfloat16)
        vb = v_ref[...].astype(jnp.bfloat16)
        wqb = wq_ref[...].astype(jnp.bfloat16)
        wob = wo_ref[...].astype(jnp.bfloat16)

        qg = lax.broadcasted_iota(jnp.int32, (M, SKV), 0) // 64
        kbm = lax.broadcasted_iota(jnp.int32, (M, SKV), 1) // 64
        mask = qg == kbm

        def compute_partial(slot):
            for b in range(B):
                xb = xg_ref[slot, b]
                q = jnp.dot(xb, wqb, preferred_element_type=jnp.float32)
                qbf = q.astype(jnp.bfloat16)
                ctxs = []
                for h in range(H):
                    qh = qbf[:, h * DH:(h + 1) * DH]
                    s = jnp.dot(qh, kb[b, :, h, :].T,
                                preferred_element_type=jnp.float32)
                    s = jnp.where(mask, s * 0.125, -1e9)
                    m = jnp.max(s, axis=-1, keepdims=True)
                    e = jnp.exp(s - m)
                    den = jnp.sum(e, axis=-1, keepdims=True)
                    w = (e / den).astype(jnp.bfloat16)
                    ctxs.append(jnp.dot(w, vb[b, :, h, :],
                                        preferred_element_type=jnp.float32))
                ctx = jnp.concatenate(ctxs, axis=-1).astype(jnp.bfloat16)
                yacc_ref[slot, b] = jnp.dot(
                    ctx, wob, preferred_element_type=jnp.float32)

        mask2 = jnp.concatenate([mask, mask], axis=0)

        def compute_partial2(slot_a, slot_b):
            for b in range(B):
                xcat = jnp.concatenate(
                    [xg_ref[slot_a, b], xg_ref[slot_b, b]], axis=0)
                q = jnp.dot(xcat, wqb, preferred_element_type=jnp.float32)
                qbf = q.astype(jnp.bfloat16)
                ctxs = []
                for h in range(H):
                    qh = qbf[:, h * DH:(h + 1) * DH]
                    s = jnp.dot(qh, kb[b, :, h, :].T,
                                preferred_element_type=jnp.float32)
                    s = jnp.where(mask2, s * 0.125, -1e9)
                    m = jnp.max(s, axis=-1, keepdims=True)
                    e = jnp.exp(s - m)
                    den = jnp.sum(e, axis=-1, keepdims=True)
                    w = (e / den).astype(jnp.bfloat16)
                    ctxs.append(jnp.dot(w, vb[b, :, h, :],
                                        preferred_element_type=jnp.float32))
                ctx = jnp.concatenate(ctxs, axis=-1).astype(jnp.bfloat16)
                y = jnp.dot(ctx, wob, preferred_element_type=jnp.float32)
                yacc_ref[slot_a, b] = y[0:M, :]
                yacc_ref[slot_b, b] = y[M:2 * M, :]

        fl_owner = jnp.mod(my + 7, N_DEV)
        fr_owner = jnp.mod(my + 8, N_DEV)
        fin_l_send = pltpu.make_async_remote_copy(
            src_ref=ysnd_l, dst_ref=yfin_l,
            send_sem=fin_ssem_l, recv_sem=fin_rsem_l,
            device_id=(fl_owner,), device_id_type=pl.DeviceIdType.MESH)
        fin_l_recv = pltpu.make_async_remote_copy(
            src_ref=ysnd_l, dst_ref=yfin_l,
            send_sem=fin_ssem_l, recv_sem=fin_rsem_l,
            device_id=(fl_owner,), device_id_type=pl.DeviceIdType.MESH)
        fin_r_send = pltpu.make_async_remote_copy(
            src_ref=ysnd_r, dst_ref=yfin_r,
            send_sem=fin_ssem_r, recv_sem=fin_rsem_r,
            device_id=(fr_owner,), device_id_type=pl.DeviceIdType.MESH)
        fin_r_recv = pltpu.make_async_remote_copy(
            src_ref=ysnd_r, dst_ref=yfin_r,
            send_sem=fin_ssem_r, recv_sem=fin_rsem_r,
            device_id=(fr_owner,), device_id_type=pl.DeviceIdType.MESH)

        for g in range(NT):
            bs_r = jnp.mod(my - g + 2 * N_DEV, N_DEV)
            br_r = jnp.mod(my - 1 - g + 2 * N_DEV, N_DEV)
            cs_r = jnp.mod(my + 1 - g + 2 * N_DEV, N_DEV)
            bs_l = jnp.mod(my + g, N_DEV)
            br_l = jnp.mod(my + 1 + g, N_DEV)
            cs_l = jnp.mod(my - 1 + g + N_DEV, N_DEV)
            tick_a = jnp.mod(bs_r, 2) == 0

            blks = []
            if g < 8:
                blks.append((pltpu.make_async_remote_copy(
                    src_ref=xg_ref.at[bs_r // 2], dst_ref=xg_ref.at[bs_r // 2],
                    send_sem=blk_ssem_r.at[g], recv_sem=blk_rsem_r.at[g],
                    device_id=(right,), device_id_type=pl.DeviceIdType.MESH),
                    pltpu.make_async_remote_copy(
                    src_ref=xg_ref.at[br_r // 2], dst_ref=xg_ref.at[br_r // 2],
                    send_sem=blk_ssem_r.at[g], recv_sem=blk_rsem_r.at[g],
                    device_id=(right,), device_id_type=pl.DeviceIdType.MESH)))
            if g < 7:
                blks.append((pltpu.make_async_remote_copy(
                    src_ref=xg_ref.at[bs_l // 2], dst_ref=xg_ref.at[bs_l // 2],
                    send_sem=blk_ssem_l.at[g], recv_sem=blk_rsem_l.at[g],
                    device_id=(left,), device_id_type=pl.DeviceIdType.MESH),
                    pltpu.make_async_remote_copy(
                    src_ref=xg_ref.at[br_l // 2], dst_ref=xg_ref.at[br_l // 2],
                    send_sem=blk_ssem_l.at[g], recv_sem=blk_rsem_l.at[g],
                    device_id=(left,), device_id_type=pl.DeviceIdType.MESH)))

            chks = []
            if 2 <= g:
                slot_r = (g - 2) // 2
                chks.append((cs_r, ysnd_r, pltpu.make_async_remote_copy(
                    src_ref=ysnd_r, dst_ref=rsbuf_r.at[slot_r],
                    send_sem=chk_ssem_r.at[g - 2],
                    recv_sem=chk_rsem_r.at[g - 2],
                    device_id=(right,), device_id_type=pl.DeviceIdType.MESH),
                    bs_r, rsbuf_r, slot_r, pltpu.make_async_remote_copy(
                    src_ref=rsbuf_r.at[slot_r], dst_ref=rsbuf_r.at[slot_r],
                    send_sem=chk_ssem_r.at[g - 2],
                    recv_sem=chk_rsem_r.at[g - 2],
                    device_id=(right,), device_id_type=pl.DeviceIdType.MESH)))
            if 2 <= g < 8:
                slot_l = (g - 2) // 2
                chks.append((cs_l, ysnd_l, pltpu.make_async_remote_copy(
                    src_ref=ysnd_l, dst_ref=rsbuf_l.at[slot_l],
                    send_sem=chk_ssem_l.at[g - 2],
                    recv_sem=chk_rsem_l.at[g - 2],
                    device_id=(left,), device_id_type=pl.DeviceIdType.MESH),
                    bs_l, rsbuf_l, slot_l, pltpu.make_async_remote_copy(
                    src_ref=rsbuf_l.at[slot_l], dst_ref=rsbuf_l.at[slot_l],
                    send_sem=chk_ssem_l.at[g - 2],
                    recv_sem=chk_rsem_l.at[g - 2],
                    device_id=(left,), device_id_type=pl.DeviceIdType.MESH)))

            if blks:
                @pl.when(tick_a)
                def _():
                    for blk_send, _ in blks:
                        blk_send.start()
            if chks:
                @pl.when(jnp.logical_not(tick_a))
                def _():
                    for cs, ysnd, chk_send, _, _, _, _ in chks:
                        ysnd[...] = yacc_ref[cs // 2].astype(jnp.bfloat16)
                        chk_send.start()
            if g == 8:
                @pl.when(jnp.logical_not(tick_a))
                def _():
                    ysnd_l[...] = yacc_ref[
                        jnp.mod(my + 7, N_DEV) // 2].astype(jnp.bfloat16)
                    fin_l_send.start()

            @pl.when(tick_a)
            def _():
                if 1 <= g < 8:
                    compute_partial2(bs_r // 2, bs_l // 2)
                else:
                    compute_partial(bs_r // 2)

            if blks:
                @pl.when(jnp.logical_not(tick_a))
                def _():
                    for _, blk_recv in blks:
                        blk_recv.wait_recv()
            if chks:
                @pl.when(tick_a)
                def _():
                    for _, _, _, cr, rsbuf, slot, chk_recv in chks:
                        chk_recv.wait_recv()
                        yacc_ref[cr // 2] = (
                            yacc_ref[cr // 2]
                            + rsbuf[slot].astype(jnp.float32))
            if blks:
                @pl.when(tick_a)
                def _():
                    for blk_send, _ in blks:
                        blk_send.wait_send()
            if chks:
                @pl.when(jnp.logical_not(tick_a))
                def _():
                    for _, _, chk_send, _, _, _, _ in chks:
                        chk_send.wait_send()

        @pl.when(jnp.logical_not(i_am_live))
        def _():
            fin_l_send.wait_send()
            out_ref[...] = jnp.zeros((B, M, F), jnp.float32)

        @pl.when(i_am_live)
        def _():
            ysnd_r[...] = yacc_ref[jnp.mod(my - 8 + N_DEV, N_DEV) // 2].astype(jnp.bfloat16)
            fin_r_send.start()
            fin_l_recv.wait_recv()
            fin_r_recv.wait_recv()
            out_ref[...] = (yacc_ref[my // 2]
                            + yfin_l[...].astype(jnp.float32)
                            + yfin_r[...].astype(jnp.float32))
            fin_r_send.wait_send()

        @functools.partial(pl.run_scoped, exit_sem=pltpu.SemaphoreType.REGULAR)
        def _(exit_sem):
            for nbr in (left, right):
                pl.semaphore_signal(
                    exit_sem, inc=1,
                    device_id=(nbr,), device_id_type=pl.DeviceIdType.MESH,
                )
            pl.semaphore_wait(exit_sem, 2)

    return pl.pallas_call(
        body,
        out_shape=jax.ShapeDtypeStruct((B, M, F), jnp.float32),
        in_specs=[pl.BlockSpec(memory_space=pltpu.VMEM)] * 5,
        out_specs=pl.BlockSpec(memory_space=pltpu.VMEM),
        scratch_shapes=[
            pltpu.VMEM((NL, B, M, F), jnp.bfloat16),
            pltpu.VMEM((NL, B, M, F), jnp.float32),
            pltpu.VMEM((4, B, M, F), jnp.bfloat16),
            pltpu.VMEM((3, B, M, F), jnp.bfloat16),
            pltpu.VMEM((B, M, F), jnp.bfloat16),
            pltpu.VMEM((B, M, F), jnp.bfloat16),
            pltpu.VMEM((B, M, F), jnp.bfloat16),
            pltpu.VMEM((B, M, F), jnp.bfloat16),
            pltpu.SemaphoreType.DMA((8,)),
            pltpu.SemaphoreType.DMA((8,)),
            pltpu.SemaphoreType.DMA((7,)),
            pltpu.SemaphoreType.DMA((7,)),
            pltpu.SemaphoreType.DMA((7,)),
            pltpu.SemaphoreType.DMA((7,)),
            pltpu.SemaphoreType.DMA((6,)),
            pltpu.SemaphoreType.DMA((6,)),
            pltpu.SemaphoreType.DMA,
            pltpu.SemaphoreType.DMA,
            pltpu.SemaphoreType.DMA,
            pltpu.SemaphoreType.DMA,
        ],
        compiler_params=pltpu.CompilerParams(collective_id=0),
    )(x, Wq, K_my, V_my, Wo)
